# baseline (speedup 1.0000x reference)
import functools

import jax
import jax.numpy as jnp
from jax import lax
from jax.experimental import pallas as pl
from jax.experimental.pallas import tpu as pltpu
from jax.experimental.pallas import tpu_sc as plsc
from jax._src.pallas import core as pallas_core
from jax._src.pallas import mpmd

VOCAB = 100000
DIM = 768
B = 256
H = 14
W = 14
HW = H * W
N = B * HW

NC = 2
NS = 16
NW = NC * NS
PER_W = N // NW
CHUNK = 32
NCHUNK = PER_W // CHUNK
NSLOT = 3

_VMESH = plsc.VectorSubcoreMesh(core_axis_name="c", subcore_axis_name="s")
_SMESH = plsc.ScalarSubcoreMesh(axis_name="c", num_cores=NC)


def _tec_fn(table, idx, out, idx_v, b0, b1, s0, s1, spm, fill, dsem, credit):
    del out, dsem
    k = lax.axis_index("c")
    s = lax.axis_index("s")
    wid = s * NC + k
    base = wid * PER_W
    pltpu.sync_copy(idx.at[pl.ds(base, PER_W)], idx_v)

    def fire(c, buf, sem):
        pltpu.async_copy(
            table.at[idx_v.at[pl.ds(c * CHUNK, CHUNK)]], buf, sem
        )

    def drain(c, buf, sem, backpressure=True):
        pltpu.make_async_copy(
            table.at[idx_v.at[pl.ds(c * CHUNK, CHUNK)]], buf, sem
        ).wait()
        if backpressure:
            pl.semaphore_wait(credit, 1)
        pltpu.sync_copy(buf, spm.at[s, c % NSLOT])
        pltpu.semaphore_signal(fill.at[s])

    fire(0, b0, s0)
    fire(1, b1, s1)

    def body(j, carry):
        c = 2 * j
        drain(c, b0, s0)
        fire(c + 2, b0, s0)
        drain(c + 1, b1, s1)
        fire(c + 3, b1, s1)
        return carry

    drain(0, b0, s0, backpressure=False)
    fire(2, b0, s0)
    drain(1, b1, s1, backpressure=False)
    fire(3, b1, s1)
    drain(2, b0, s0, backpressure=False)
    fire(4, b0, s0)
    drain(3, b1, s1)
    fire(5, b1, s1)
    lax.fori_loop(2, (NCHUNK - 3) // 2, body, 0)
    drain(NCHUNK - 3, b0, s0)
    fire(NCHUNK - 1, b0, s0)
    drain(NCHUNK - 2, b1, s1)
    drain(NCHUNK - 1, b0, s0)


def _scs_fn(table, idx, out, idx_v, b0, b1, s0, s1, spm, fill, dsem, credit):
    del table, idx, idx_v, b0, b1, s0, s1
    k = lax.axis_index("c")

    def round_body(c, carry):
        def issue(t, _):
            pl.semaphore_wait(fill.at[t], 1)
            base = (t * NC + k) * PER_W + c * CHUNK
            pltpu.async_copy(spm.at[t, c % NSLOT], out.at[pl.ds(base, CHUNK)], dsem)
            return _

        lax.fori_loop(0, NS, issue, 0)

        def drain(t, _):
            base = (t * NC + k) * PER_W + c * CHUNK
            pltpu.make_async_copy(
                spm.at[t, c % NSLOT], out.at[pl.ds(base, CHUNK)], dsem
            ).wait()
            pltpu.semaphore_signal(credit, 1, device_id={"s": t})
            return _

        lax.fori_loop(0, NS, drain, 0)
        return carry

    lax.fori_loop(0, NCHUNK, round_body, 0)


@jax.jit
def _sc_gather(token_emb, ids):
    return mpmd.mpmd_map(
        [(_SMESH, _scs_fn), (_VMESH, _tec_fn)],
        out_types=[jax.ShapeDtypeStruct((N, DIM), jnp.float32)],
        scratch_types=[
            pallas_core.CoreMemorySpace(pltpu.VMEM, _VMESH)((PER_W,), jnp.int32),
            pallas_core.CoreMemorySpace(pltpu.VMEM, _VMESH)((CHUNK, DIM), jnp.float32),
            pallas_core.CoreMemorySpace(pltpu.VMEM, _VMESH)((CHUNK, DIM), jnp.float32),
            pltpu.SemaphoreType.DMA @ _VMESH,
            pltpu.SemaphoreType.DMA @ _VMESH,
            pltpu.VMEM_SHARED((NS, NSLOT, CHUNK, DIM), jnp.float32),
            pallas_core.CoreMemorySpace(pltpu.SEMAPHORE, _SMESH)(
                (NS,), pltpu.SemaphoreType.REGULAR.dtype
            ),
            pltpu.SemaphoreType.DMA @ _SMESH,
            pltpu.SemaphoreType.REGULAR @ _VMESH,
        ],
    )(token_emb, ids)[0]


def _emb_body(pos_ref, mod_ref, out_ref):
    out_ref[...] = jnp.broadcast_to(
        pos_ref[0][:, None, :] + mod_ref[...], out_ref.shape
    )


def _build_2d_sincos_posemb(h, w, embed_dim, temperature=10000.0):
    grid_w = jnp.arange(w, dtype=jnp.float32)
    grid_h = jnp.arange(h, dtype=jnp.float32)
    grid_w, grid_h = jnp.meshgrid(grid_w, grid_h, indexing='ij')
    pos_dim = embed_dim // 4
    omega = jnp.arange(pos_dim, dtype=jnp.float32) / pos_dim
    omega = 1.0 / (temperature ** omega)
    out_w = jnp.einsum('m,d->md', grid_w.flatten(), omega)
    out_h = jnp.einsum('m,d->md', grid_h.flatten(), omega)
    return jnp.concatenate(
        [jnp.sin(out_w), jnp.cos(out_w), jnp.sin(out_h), jnp.cos(out_h)],
        axis=1,
    )


_EMB_BP = 14


@jax.jit
def _tc_emb(pos, mod):
    return pl.pallas_call(
        _emb_body,
        grid=(HW // _EMB_BP,),
        in_specs=[
            pl.BlockSpec((1, _EMB_BP, DIM), lambda i: (i, 0, 0)),
            pl.BlockSpec((1, 1, DIM), lambda i: (0, 0, 0)),
        ],
        out_specs=pl.BlockSpec((_EMB_BP, B, DIM), lambda i: (i, 0, 0)),
        out_shape=jax.ShapeDtypeStruct((HW, B, DIM), jnp.float32),
    )(pos.reshape(HW // _EMB_BP, _EMB_BP, DIM), mod)


def kernel(tensor, token_emb, mod_emb):
    ids_t = tensor.reshape(B, HW).astype(jnp.int32).T.reshape(N)
    x_flat = _sc_gather(token_emb, ids_t)
    pos = _build_2d_sincos_posemb(H, W, DIM)
    emb_t = _tc_emb(pos, mod_emb)
    x = jnp.transpose(x_flat.reshape(HW, B, DIM), (1, 0, 2))
    emb = jnp.transpose(emb_t, (1, 0, 2))
    return (x, emb)
